# Initial kernel scaffold; baseline (speedup 1.0000x reference)
import jax
import jax.numpy as jnp
from jax.experimental import pallas as pl


def kernel(x1, W_up, b_up, W_c1, b_c1, gamma1, beta1, W_c2, b_c2, gamma2, beta2, upconv_top_index, upconv_down_index, neigh_orders):
    raise NotImplementedError("write your pallas kernel here")


# speedup vs baseline: 2.4554x; 2.4554x over previous
import functools

import jax
import jax.numpy as jnp
from jax import lax
from jax.experimental import pallas as pl
from jax.experimental.pallas import tpu as pltpu
from jax.experimental.pallas import tpu_sc as plsc

RAW = 2562
NEW = 10242
C = 256
K7 = 7 * C
IN_CH = 512

TOP_PAD = 2688
DOWN = 7680
NPAD = 10752
SHIFT = TOP_PAD - RAW
B3 = 7 * NPAD
NW = 32
CHUNK = 112

M1 = 2688
MBLK = 672
NBLK = NPAD // MBLK


def _make_sc_gather(T, B, D=C):
    bpw = B // NW
    nch = bpw // CHUNK
    mesh = plsc.VectorSubcoreMesh(core_axis_name="c", subcore_axis_name="s")

    def body(table, idx, out, idx_v, rows_v, sem):
        cc = lax.axis_index("c")
        ss = lax.axis_index("s")
        wid = ss * 2 + cc
        base0 = wid * bpw
        for k in range(nch):
            base = pl.multiple_of(base0 + k * CHUNK, 8)
            pltpu.sync_copy(idx.at[pl.ds(base, CHUNK)], idx_v)
            pltpu.async_copy(table.at[idx_v], rows_v, sem).wait()
            pltpu.sync_copy(rows_v, out.at[pl.ds(base, CHUNK)])

    return pl.kernel(
        body,
        mesh=mesh,
        out_type=jax.ShapeDtypeStruct((B, D), jnp.float32),
        scratch_types=[
            pltpu.VMEM((CHUNK,), jnp.int32),
            pltpu.VMEM((CHUNK, D), jnp.float32),
            pltpu.SemaphoreType.DMA,
        ],
    )


def _up_mm_body(x_ref, w_ref, b_ref, o_ref):
    o_ref[...] = (
        jnp.dot(x_ref[...], w_ref[...], preferred_element_type=jnp.float32)
        + b_ref[...]
    )


def _assemble_body(ge_ref, go_ref, sl_ref, sr_ref, o_ref):
    i = pl.program_id(0)

    @pl.when(i < TOP_PAD // MBLK)
    def _top():
        o_ref[...] = ge_ref[...]

    @pl.when(i >= TOP_PAD // MBLK)
    def _down():
        o_ref[...] = jnp.dot(
            ge_ref[...], sl_ref[...], preferred_element_type=jnp.float32
        ) + jnp.dot(go_ref[...], sr_ref[...], preferred_element_type=jnp.float32)


def _conv_mm_body(g_ref, w_ref, b_ref, z_ref, st_ref, acc_ref):
    i = pl.program_id(0)
    z = (
        jnp.dot(g_ref[...], w_ref[...], preferred_element_type=jnp.float32)
        + b_ref[...]
    )
    z_ref[...] = z
    rows = i * MBLK + lax.broadcasted_iota(jnp.int32, (MBLK, 1), 0)
    zm = jnp.where(rows < NEW, z, 0.0)

    @pl.when(i == 0)
    def _init():
        acc_ref[...] = jnp.zeros_like(acc_ref)

    acc_ref[0:1, :] += jnp.sum(zm, axis=0, keepdims=True)
    acc_ref[1:2, :] += jnp.sum(zm * zm, axis=0, keepdims=True)

    @pl.when(i == NBLK - 1)
    def _fin():
        st_ref[...] = acc_ref[...]


def _bn_act_body(z_ref, st_ref, gam_ref, bet_ref, o_ref):
    inv_n = 1.0 / NEW
    mean = st_ref[0:1, :] * inv_n
    var = st_ref[1:2, :] * inv_n - mean * mean
    scale = gam_ref[...] * lax.rsqrt(var + 1e-5)
    shift = bet_ref[...] - mean * scale
    a = z_ref[...] * scale + shift
    o_ref[...] = jnp.where(a >= 0, a, 0.2 * a)


def _up_matmul(x1p, W_up, b_up):
    return pl.pallas_call(
        _up_mm_body,
        grid=(7,),
        in_specs=[
            pl.BlockSpec((M1, IN_CH), lambda j: (0, 0)),
            pl.BlockSpec((IN_CH, C), lambda j: (0, j)),
            pl.BlockSpec((1, C), lambda j: (0, j)),
        ],
        out_specs=pl.BlockSpec((M1, C), lambda j: (0, j)),
        out_shape=jax.ShapeDtypeStruct((M1, K7), jnp.float32),
    )(x1p, W_up, b_up.reshape(1, K7))


def _assemble_x(ge, go, sl, sr):
    return pl.pallas_call(
        _assemble_body,
        grid=(NBLK,),
        in_specs=[
            pl.BlockSpec((MBLK, C), lambda i: (i, 0)),
            pl.BlockSpec((MBLK, C), lambda i: (i, 0)),
            pl.BlockSpec((C, C), lambda i: (0, 0)),
            pl.BlockSpec((C, C), lambda i: (0, 0)),
        ],
        out_specs=pl.BlockSpec((MBLK, C), lambda i: (i, 0)),
        out_shape=jax.ShapeDtypeStruct((NPAD, C), jnp.float32),
    )(ge, go, sl, sr)


def _conv_matmul(g, W, b):
    return pl.pallas_call(
        _conv_mm_body,
        grid=(NBLK,),
        in_specs=[
            pl.BlockSpec((MBLK, K7), lambda i: (i, 0)),
            pl.BlockSpec((K7, C), lambda i: (0, 0)),
            pl.BlockSpec((1, C), lambda i: (0, 0)),
        ],
        out_specs=[
            pl.BlockSpec((MBLK, C), lambda i: (i, 0)),
            pl.BlockSpec((2, C), lambda i: (0, 0)),
        ],
        out_shape=[
            jax.ShapeDtypeStruct((NPAD, C), jnp.float32),
            jax.ShapeDtypeStruct((2, C), jnp.float32),
        ],
        scratch_shapes=[pltpu.VMEM((2, C), jnp.float32)],
    )(g.reshape(NPAD, K7), W, b.reshape(1, C))


def _bn_act(z, st, gamma, beta):
    return pl.pallas_call(
        _bn_act_body,
        grid=(NBLK,),
        in_specs=[
            pl.BlockSpec((MBLK, C), lambda i: (i, 0)),
            pl.BlockSpec((2, C), lambda i: (0, 0)),
            pl.BlockSpec((1, C), lambda i: (0, 0)),
            pl.BlockSpec((1, C), lambda i: (0, 0)),
        ],
        out_specs=pl.BlockSpec((MBLK, C), lambda i: (i, 0)),
        out_shape=jax.ShapeDtypeStruct((NPAD, C), jnp.float32),
    )(z, st, gamma.reshape(1, C), beta.reshape(1, C))


def _sc_gather(table, idx, B):
    return _make_sc_gather(table.shape[0], B)(table, idx)


def kernel(x1, W_up, b_up, W_c1, b_c1, gamma1, beta1, W_c2, b_c2, gamma2,
           beta2, upconv_top_index, upconv_down_index, neigh_orders):
    i32 = jnp.int32
    top = upconv_top_index.astype(i32)
    dn = upconv_down_index.astype(i32).reshape(-1, 2)
    neigh = neigh_orders.astype(i32)

    zpad_top = jnp.zeros((SHIFT,), i32)
    zpad_dn = jnp.zeros((NPAD - TOP_PAD - DOWN,), i32)
    eidx = jnp.concatenate([top, zpad_top, dn[:, 0], zpad_dn])
    oidx = jnp.concatenate([top, zpad_top, dn[:, 1], zpad_dn])

    neigh1 = jnp.where(neigh >= RAW, neigh + SHIFT, neigh)
    zpad_g = jnp.zeros((B3 - 7 * NEW,), i32)
    nidx1 = jnp.concatenate([neigh1, zpad_g])
    nidx2 = jnp.concatenate([neigh, zpad_g])

    ccol = jnp.arange(C)[:, None] // 2
    krow = jnp.arange(C)[None, :]
    sl = jnp.where(ccol == krow, 0.5, 0.0).astype(jnp.float32)
    sr = jnp.where(ccol == (krow - 128), 0.5, 0.0).astype(jnp.float32)

    x1p = jnp.pad(x1, ((0, M1 - RAW), (0, 0)))

    up_flat = _up_matmul(x1p, W_up, b_up).reshape(M1 * 7, C)

    ge = _sc_gather(up_flat, eidx, NPAD)
    go = _sc_gather(up_flat, oidx, NPAD)
    x = _assemble_x(ge, go, sl, sr)

    g1 = _sc_gather(x, nidx1, B3)
    z1, st1 = _conv_matmul(g1, W_c1, b_c1)
    a1 = _bn_act(z1, st1, gamma1, beta1)

    g2 = _sc_gather(a1, nidx2, B3)
    z2, st2 = _conv_matmul(g2, W_c2, b_c2)
    h2 = _bn_act(z2, st2, gamma2, beta2)

    return h2[:NEW]


# speedup vs baseline: 2.6085x; 1.0623x over previous
import functools

import jax
import jax.numpy as jnp
from jax import lax
from jax.experimental import pallas as pl
from jax.experimental.pallas import tpu as pltpu
from jax.experimental.pallas import tpu_sc as plsc

RAW = 2562
NEW = 10242
C = 256
K7 = 7 * C
IN_CH = 512

TOP_PAD = 2688
DOWN = 7680
NPAD = 10752
SHIFT = TOP_PAD - RAW
B3 = 7 * NPAD
NW = 32
CHUNK = 112

M1 = 2688
MBLK = 672
NBLK = NPAD // MBLK


NBUF = 3


def _make_sc_gather(T, B, D=C):
    bpw = B // NW
    nch = bpw // CHUNK
    mesh = plsc.VectorSubcoreMesh(core_axis_name="c", subcore_axis_name="s")

    def body(table, idx, out, idx_v, b0, b1, b2, g0, g1, g2, w0, w1, w2):
        bufs = (b0, b1, b2)
        gsems = (g0, g1, g2)
        wsems = (w0, w1, w2)
        cc = lax.axis_index("c")
        ss = lax.axis_index("s")
        wid = ss * 2 + cc
        base0 = pl.multiple_of(wid * bpw, 8)
        pltpu.sync_copy(idx.at[pl.ds(base0, bpw)], idx_v)
        gh = [None] * nch
        wh = [None] * nch
        for k in range(nch):
            b = k % NBUF
            if k >= NBUF:
                wh[k - NBUF].wait()
            gh[k] = pltpu.async_copy(
                table.at[idx_v.at[pl.ds(k * CHUNK, CHUNK)]], bufs[b], gsems[b]
            )
            if k >= 1:
                pb = (k - 1) % NBUF
                gh[k - 1].wait()
                wh[k - 1] = pltpu.async_copy(
                    bufs[pb],
                    out.at[pl.ds(pl.multiple_of(base0 + (k - 1) * CHUNK, 8), CHUNK)],
                    wsems[pb],
                )
        gh[nch - 1].wait()
        lb = (nch - 1) % NBUF
        wh[nch - 1] = pltpu.async_copy(
            bufs[lb],
            out.at[pl.ds(pl.multiple_of(base0 + (nch - 1) * CHUNK, 8), CHUNK)],
            wsems[lb],
        )
        for k in range(max(0, nch - NBUF), nch):
            wh[k].wait()

    return pl.kernel(
        body,
        mesh=mesh,
        out_type=jax.ShapeDtypeStruct((B, D), jnp.float32),
        scratch_types=[
            pltpu.VMEM((bpw,), jnp.int32),
            pltpu.VMEM((CHUNK, D), jnp.float32),
            pltpu.VMEM((CHUNK, D), jnp.float32),
            pltpu.VMEM((CHUNK, D), jnp.float32),
            pltpu.SemaphoreType.DMA,
            pltpu.SemaphoreType.DMA,
            pltpu.SemaphoreType.DMA,
            pltpu.SemaphoreType.DMA,
            pltpu.SemaphoreType.DMA,
            pltpu.SemaphoreType.DMA,
        ],
    )


def _up_mm_body(x_ref, w_ref, b_ref, o_ref):
    o_ref[...] = (
        jnp.dot(x_ref[...], w_ref[...], preferred_element_type=jnp.float32)
        + b_ref[...]
    )


def _assemble_body(ge_ref, go_ref, sl_ref, sr_ref, o_ref):
    i = pl.program_id(0)

    @pl.when(i < TOP_PAD // MBLK)
    def _top():
        o_ref[...] = ge_ref[...]

    @pl.when(i >= TOP_PAD // MBLK)
    def _down():
        o_ref[...] = jnp.dot(
            ge_ref[...], sl_ref[...], preferred_element_type=jnp.float32
        ) + jnp.dot(go_ref[...], sr_ref[...], preferred_element_type=jnp.float32)


def _conv_mm_body(g_ref, w_ref, b_ref, z_ref, st_ref, acc_ref):
    i = pl.program_id(0)
    z = (
        jnp.dot(g_ref[...], w_ref[...], preferred_element_type=jnp.float32)
        + b_ref[...]
    )
    z_ref[...] = z
    rows = i * MBLK + lax.broadcasted_iota(jnp.int32, (MBLK, 1), 0)
    zm = jnp.where(rows < NEW, z, 0.0)

    @pl.when(i == 0)
    def _init():
        acc_ref[...] = jnp.zeros_like(acc_ref)

    acc_ref[0:1, :] += jnp.sum(zm, axis=0, keepdims=True)
    acc_ref[1:2, :] += jnp.sum(zm * zm, axis=0, keepdims=True)

    @pl.when(i == NBLK - 1)
    def _fin():
        st_ref[...] = acc_ref[...]


def _bn_act_body(z_ref, st_ref, gam_ref, bet_ref, o_ref):
    inv_n = 1.0 / NEW
    mean = st_ref[0:1, :] * inv_n
    var = st_ref[1:2, :] * inv_n - mean * mean
    scale = gam_ref[...] * lax.rsqrt(var + 1e-5)
    shift = bet_ref[...] - mean * scale
    a = z_ref[...] * scale + shift
    o_ref[...] = jnp.where(a >= 0, a, 0.2 * a)


def _up_matmul(x1p, W_up, b_up):
    return pl.pallas_call(
        _up_mm_body,
        grid=(7,),
        in_specs=[
            pl.BlockSpec((M1, IN_CH), lambda j: (0, 0)),
            pl.BlockSpec((IN_CH, C), lambda j: (0, j)),
            pl.BlockSpec((1, C), lambda j: (0, j)),
        ],
        out_specs=pl.BlockSpec((M1, C), lambda j: (0, j)),
        out_shape=jax.ShapeDtypeStruct((M1, K7), jnp.float32),
    )(x1p, W_up, b_up.reshape(1, K7))


def _assemble_x(ge, go, sl, sr):
    return pl.pallas_call(
        _assemble_body,
        grid=(NBLK,),
        in_specs=[
            pl.BlockSpec((MBLK, C), lambda i: (i, 0)),
            pl.BlockSpec((MBLK, C), lambda i: (i, 0)),
            pl.BlockSpec((C, C), lambda i: (0, 0)),
            pl.BlockSpec((C, C), lambda i: (0, 0)),
        ],
        out_specs=pl.BlockSpec((MBLK, C), lambda i: (i, 0)),
        out_shape=jax.ShapeDtypeStruct((NPAD, C), jnp.float32),
    )(ge, go, sl, sr)


def _conv_matmul(g, W, b):
    return pl.pallas_call(
        _conv_mm_body,
        grid=(NBLK,),
        in_specs=[
            pl.BlockSpec((MBLK, K7), lambda i: (i, 0)),
            pl.BlockSpec((K7, C), lambda i: (0, 0)),
            pl.BlockSpec((1, C), lambda i: (0, 0)),
        ],
        out_specs=[
            pl.BlockSpec((MBLK, C), lambda i: (i, 0)),
            pl.BlockSpec((2, C), lambda i: (0, 0)),
        ],
        out_shape=[
            jax.ShapeDtypeStruct((NPAD, C), jnp.float32),
            jax.ShapeDtypeStruct((2, C), jnp.float32),
        ],
        scratch_shapes=[pltpu.VMEM((2, C), jnp.float32)],
    )(g.reshape(NPAD, K7), W, b.reshape(1, C))


def _bn_act(z, st, gamma, beta):
    return pl.pallas_call(
        _bn_act_body,
        grid=(NBLK,),
        in_specs=[
            pl.BlockSpec((MBLK, C), lambda i: (i, 0)),
            pl.BlockSpec((2, C), lambda i: (0, 0)),
            pl.BlockSpec((1, C), lambda i: (0, 0)),
            pl.BlockSpec((1, C), lambda i: (0, 0)),
        ],
        out_specs=pl.BlockSpec((MBLK, C), lambda i: (i, 0)),
        out_shape=jax.ShapeDtypeStruct((NPAD, C), jnp.float32),
    )(z, st, gamma.reshape(1, C), beta.reshape(1, C))


def _sc_gather(table, idx, B):
    return _make_sc_gather(table.shape[0], B)(table, idx)


def kernel(x1, W_up, b_up, W_c1, b_c1, gamma1, beta1, W_c2, b_c2, gamma2,
           beta2, upconv_top_index, upconv_down_index, neigh_orders):
    i32 = jnp.int32
    top = upconv_top_index.astype(i32)
    dn = upconv_down_index.astype(i32).reshape(-1, 2)
    neigh = neigh_orders.astype(i32)

    zpad_top = jnp.zeros((SHIFT,), i32)
    zpad_dn = jnp.zeros((NPAD - TOP_PAD - DOWN,), i32)
    eidx = jnp.concatenate([top, zpad_top, dn[:, 0], zpad_dn])
    oidx = jnp.concatenate([top, zpad_top, dn[:, 1], zpad_dn])

    neigh1 = jnp.where(neigh >= RAW, neigh + SHIFT, neigh)
    zpad_g = jnp.zeros((B3 - 7 * NEW,), i32)
    nidx1 = jnp.concatenate([neigh1, zpad_g])
    nidx2 = jnp.concatenate([neigh, zpad_g])

    ccol = jnp.arange(C)[:, None] // 2
    krow = jnp.arange(C)[None, :]
    sl = jnp.where(ccol == krow, 0.5, 0.0).astype(jnp.float32)
    sr = jnp.where(ccol == (krow - 128), 0.5, 0.0).astype(jnp.float32)

    x1p = jnp.pad(x1, ((0, M1 - RAW), (0, 0)))

    up_flat = _up_matmul(x1p, W_up, b_up).reshape(M1 * 7, C)

    ge = _sc_gather(up_flat, eidx, NPAD)
    go = _sc_gather(up_flat, oidx, NPAD)
    x = _assemble_x(ge, go, sl, sr)

    g1 = _sc_gather(x, nidx1, B3)
    z1, st1 = _conv_matmul(g1, W_c1, b_c1)
    a1 = _bn_act(z1, st1, gamma1, beta1)

    g2 = _sc_gather(a1, nidx2, B3)
    z2, st2 = _conv_matmul(g2, W_c2, b_c2)
    h2 = _bn_act(z2, st2, gamma2, beta2)

    return h2[:NEW]
